# baseline (speedup 1.0000x reference)
import functools

import jax
import jax.numpy as jnp
from jax import lax
from jax.experimental import pallas as pl
from jax.experimental.pallas import tpu as pltpu
from jax.experimental.pallas import tpu_sc as plsc

N = 10000
D = 128
DE = 16
NSLOTS = 4
NG = 16
DU = 64

NCORES = 2
NSUB = 16
NWORK = NCORES * NSUB
CHUNK = 128
CPW = 80
EPAD = NWORK * CPW * CHUNK
NPAD = 10112
RPS = NPAD // NSUB
DUMMY = N
TBLK = 128
IBLK = 8
EBLK = 1024
ZROWS = 16 - 1 - NSLOTS


def _sc_edge_pass(include_static):
    mesh = plsc.VectorSubcoreMesh(core_axis_name="c", subcore_axis_name="s")

    out_type = [jax.ShapeDtypeStruct((NCORES, NPAD, D), jnp.float32)]
    scratch = [
        pltpu.VMEM((IBLK, CHUNK), jnp.int32),
        pltpu.VMEM((IBLK, CHUNK), jnp.int32),
        pltpu.VMEM((CHUNK, D), jnp.float32),
        pltpu.VMEM((CHUNK, D), jnp.float32),
        pltpu.VMEM_SHARED((NPAD, D), jnp.float32),
        pltpu.SemaphoreType.DMA,
        pltpu.SemaphoreType.DMA,
    ]
    if include_static:
        out_type += [jax.ShapeDtypeStruct((NCORES, NPAD, D), jnp.float32)]

    def body(*refs):
        if include_static:
            (x_hbm, src_hbm, dst_hbm, est_hbm, z128_hbm,
             nbr_out, st_out,
             src_v, dst_v, buf0, buf1, acc, sem0, sem1) = refs
        else:
            (x_hbm, src_hbm, dst_hbm, z128_hbm, nbr_out,
             src_v, dst_v, buf0, buf1, acc, sem0, sem1) = refs

        c = lax.axis_index("c")
        s = lax.axis_index("s")
        w = c * NSUB + s
        base = w * CPW
        bufs = (buf0, buf1)
        sems = (sem0, sem1)

        def zero_acc():
            pltpu.sync_copy(z128_hbm.at[pl.ds(s * RPS, RPS)],
                            acc.at[pl.ds(s * RPS, RPS)])

        def writeout(out_ref):
            pltpu.sync_copy(acc.at[pl.ds(s * RPS, RPS)],
                            out_ref.at[c, pl.ds(s * RPS, RPS)])

        def edge_loop(load_src):
            @pl.loop(0, CPW // IBLK)
            def _(gb):
                pltpu.sync_copy(src_hbm.at[pl.ds(base + gb * IBLK, IBLK)],
                                src_v)
                pltpu.sync_copy(dst_hbm.at[pl.ds(base + gb * IBLK, IBLK)],
                                dst_v)
                load_src(gb, 0, buf0, sem0).start()
                for j in range(IBLK):
                    b = j % 2
                    load_src(gb, j, bufs[b], sems[b]).wait()
                    if j + 1 < IBLK:
                        load_src(gb, j + 1, bufs[1 - b], sems[1 - b]).start()
                    pltpu.sync_copy(bufs[b], acc.at[dst_v.at[j]], add=True)

        def gather_x(gb, j, buf, sem):
            return pltpu.make_async_copy(x_hbm.at[src_v.at[j]], buf, sem)

        zero_acc()
        plsc.subcore_barrier()
        edge_loop(gather_x)
        plsc.subcore_barrier()
        writeout(nbr_out)

        if include_static:
            def load_static(gb, j, buf, sem):
                row0 = (base + gb * IBLK + j) * CHUNK
                return pltpu.make_async_copy(
                    est_hbm.at[pl.ds(row0, CHUNK)], buf, sem)

            zero_acc()
            plsc.subcore_barrier()
            edge_loop(load_static)
            plsc.subcore_barrier()
            writeout(st_out)

    return pl.kernel(body, out_type=out_type, mesh=mesh, scratch_types=scratch)


def _sc_pass_full(*args):
    return _sc_edge_pass(True)(*args)


def _sc_pass_nbr(*args):
    return _sc_edge_pass(False)(*args)


def _est_body(srep, ea8, out):
    hi = jax.lax.Precision.HIGHEST
    ci = lax.broadcasted_iota(jnp.int32, (EBLK // 8, DE), 1)
    rr = lax.broadcasted_iota(jnp.int32, (DE, D), 0)
    cc = lax.broadcasted_iota(jnp.int32, (DE, D), 1)
    place0 = (cc == rr).astype(jnp.float32)
    place1 = (cc == rr + DE).astype(jnp.float32)
    r2 = lax.broadcasted_iota(jnp.int32, (D, DE), 0)
    c2 = lax.broadcasted_iota(jnp.int32, (D, DE), 1)
    for k in range(8):
        sel = (r2 == c2 + DE * k).astype(jnp.float32)
        sv = jnp.dot(srep[...], sel, precision=hi)
        ea = jnp.dot(ea8[...], sel, precision=hi)
        meta = jnp.where(ci == 0, 1.0,
                         jnp.where((ci >= 1) & (ci <= NSLOTS),
                                   (sv == (ci - 1).astype(jnp.float32))
                                   .astype(jnp.float32), 0.0))
        out[:, k, :] = (jnp.dot(meta, place0, precision=hi)
                        + jnp.dot(ea, place1, precision=hi))


def _build_est(slot_rep, ea8):
    row = lambda i: (i, 0)
    nr = EBLK // 8
    return pl.pallas_call(
        _est_body,
        grid=(EPAD // EBLK,),
        in_specs=[pl.BlockSpec((nr, D), row),
                  pl.BlockSpec((nr, D), row)],
        out_specs=pl.BlockSpec((nr, 8, D), lambda i: (i, 0, 0)),
        out_shape=jax.ShapeDtypeStruct((EPAD // 8, 8, D), jnp.float32),
    )(slot_rep, ea8).reshape(EPAD, D)


def _dup_body(x, out):
    out[...] = jnp.broadcast_to(x[...], (NCORES,) + x.shape)


def _dup_rows(x_pad):
    return pl.pallas_call(
        _dup_body,
        grid=(NPAD // TBLK,),
        in_specs=[pl.BlockSpec((TBLK, D), lambda i: (i, 0))],
        out_specs=pl.BlockSpec((NCORES, TBLK, D), lambda i: (0, i, 0)),
        out_shape=jax.ShapeDtypeStruct((NCORES, NPAD, D), jnp.float32),
    )(x_pad).reshape(NCORES * NPAD, D)


def _prep_body(u, Wm1, agg1, S1, bm1, wu1, Wm2, agg2, S2, bm2, wu2,
               B1o, T1o, B2o, T2o):
    hi = jax.lax.Precision.HIGHEST

    def one(Wm, agg, S, bm, wu, Bo, To):
        Bo[...] = jnp.dot(Wm[:D], agg[...], precision=hi)
        To[...] = jnp.concatenate([
            jnp.dot(bm[...], agg[...], precision=hi),
            jnp.dot(S[...], agg[...], precision=hi),
            jnp.zeros((ZROWS, D), jnp.float32),
            jnp.dot(Wm[D:], agg[...], precision=hi),
            jnp.dot(u[...], wu[...], precision=hi),
        ], axis=0)

    one(Wm1, agg1, S1, bm1, wu1, B1o, T1o)
    one(Wm2, agg2, S2, bm2, wu2, B2o, T2o)


def _node_body(has_residual, dup, x, nbrp, stp, batch, A, B, T, bias, res,
               out):
    hi = jax.lax.Precision.HIGHEST
    nbr = nbrp[0] + nbrp[1]
    st = stp[0] + stp[1]
    oh = (batch[...] == lax.broadcasted_iota(jnp.int32, (TBLK, NG), 1))
    z = jnp.concatenate([st[:, :2 * DE], oh.astype(jnp.float32)], axis=1)
    h = (jnp.dot(x[...], A[...], precision=hi)
         + jnp.dot(nbr, B[...], precision=hi)
         + jnp.dot(z, T[...], precision=hi)
         + bias[...])
    if has_residual:
        h = h + res[...]
    h = jnp.maximum(h, 0.0)
    if dup:
        out[...] = jnp.broadcast_to(h, (NCORES, TBLK, D))
    else:
        out[...] = h


def _node_layer(x_pad, nbr_p, st_p, batch_pad, A, B, T, bias, res_pad,
                has_residual, dup=False):
    nblk = NPAD // TBLK
    row = lambda i: (i, 0)
    full3 = pl.BlockSpec((NCORES, TBLK, D), lambda i: (0, i, 0))
    const = lambda shp: pl.BlockSpec(shp, lambda i: (0, 0))
    if dup:
        out_spec = pl.BlockSpec((NCORES, TBLK, D), lambda i: (0, i, 0))
        out_shape = jax.ShapeDtypeStruct((NCORES, NPAD, D), jnp.float32)
    else:
        out_spec = pl.BlockSpec((TBLK, D), row)
        out_shape = jax.ShapeDtypeStruct((NPAD, D), jnp.float32)
    return pl.pallas_call(
        functools.partial(_node_body, has_residual, dup),
        grid=(nblk,),
        in_specs=[
            pl.BlockSpec((TBLK, D), row),
            full3,
            full3,
            pl.BlockSpec((TBLK, 1), row),
            const((D, D)),
            const((D, D)),
            const((3 * 16, D)),
            const((1, D)),
            pl.BlockSpec((TBLK, D), row),
        ],
        out_specs=out_spec,
        out_shape=out_shape,
    )(x_pad, nbr_p, st_p, batch_pad, A, B, T, bias, res_pad)


def kernel(x, edge_index, edge_slot, edge_attr, u, batch,
           Wm1, bm1, S1, Wn1, bn1, Wm2, bm2, S2, Wn2, bn2):
    f32 = jnp.float32
    src = edge_index[0].astype(jnp.int32)
    dst = edge_index[1].astype(jnp.int32)
    slot = edge_slot.astype(jnp.int32)

    E = src.shape[0]
    pad_e = EPAD - E
    core_of_row = (jnp.arange(EPAD // CHUNK, dtype=jnp.int32)
                   // (CPW * NSUB))[:, None]
    src2d = (jnp.pad(src, (0, pad_e)).reshape(EPAD // CHUNK, CHUNK)
             + core_of_row * NPAD)
    dst2d = jnp.pad(dst, (0, pad_e), constant_values=DUMMY).reshape(
        EPAD // CHUNK, CHUNK)
    ea8 = jnp.pad(edge_attr.reshape(E // 8, D), ((0, pad_e // 8), (0, 0)))
    slot_rep = jnp.pad(
        jnp.broadcast_to(slot.astype(f32).reshape(E // 8, 8, 1),
                         (E // 8, 8, DE)).reshape(E // 8, D),
        ((0, pad_e // 8), (0, 0)))
    est128 = _build_est(slot_rep, ea8)

    x_pad = jnp.pad(x, ((0, NPAD - N), (0, 0)))
    batch_pad = jnp.pad(batch.astype(jnp.int32), (0, NPAD - N)).reshape(
        NPAD, 1)
    z128 = jnp.zeros((NPAD, D), f32)

    wshape = jax.ShapeDtypeStruct((D, D), f32)
    tshape = jax.ShapeDtypeStruct((3 * 16, D), f32)
    B1, T1, B2, T2 = pl.pallas_call(
        _prep_body,
        out_shape=[wshape, tshape, wshape, tshape],
    )(u, Wm1, Wn1[D:2 * D], S1, bm1.reshape(1, D), Wn1[2 * D:],
      Wm2, Wn2[D:2 * D], S2, bm2.reshape(1, D), Wn2[2 * D:])

    x_dup = _dup_rows(x_pad)
    nbr1_p, st_p = _sc_pass_full(x_dup, src2d, dst2d, est128, z128)

    h1_dup = _node_layer(x_pad, nbr1_p, st_p, batch_pad,
                         Wn1[:D], B1, T1, bn1.reshape(1, D), x_pad, False,
                         dup=True)

    (nbr2_p,) = _sc_pass_nbr(h1_dup.reshape(NCORES * NPAD, D), src2d, dst2d,
                             z128)

    out = _node_layer(h1_dup[0], nbr2_p, st_p, batch_pad,
                      Wn2[:D], B2, T2, bn2.reshape(1, D), x_pad, True)

    return out[:N]


# speedup vs baseline: 1.3081x; 1.3081x over previous
import functools

import jax
import jax.numpy as jnp
from jax import lax
from jax.experimental import pallas as pl
from jax.experimental.pallas import tpu as pltpu
from jax.experimental.pallas import tpu_sc as plsc

N = 10000
D = 128
DE = 16
NSLOTS = 4
NG = 16
DU = 64

NCORES = 2
NSUB = 16
NWORK = NCORES * NSUB
CHUNK = 128
CPW = 80
EPAD = NWORK * CPW * CHUNK
NPAD = 10112
RPS = NPAD // NSUB
DUMMY = N
TBLK = 128
IBLK = 8
EBLK = 1024
ZROWS = 16 - 1 - NSLOTS


def _sc_edge_pass(include_static):
    mesh = plsc.VectorSubcoreMesh(core_axis_name="c", subcore_axis_name="s")

    out_type = [jax.ShapeDtypeStruct((NCORES, NPAD, D), jnp.float32)]
    scratch = [
        pltpu.VMEM((IBLK, CHUNK), jnp.int32),
        pltpu.VMEM((IBLK, CHUNK), jnp.int32),
        pltpu.VMEM((CHUNK, D), jnp.float32),
        pltpu.VMEM((CHUNK, D), jnp.float32),
        pltpu.VMEM_SHARED((NPAD, D), jnp.float32),
        pltpu.SemaphoreType.DMA,
        pltpu.SemaphoreType.DMA,
    ]
    if include_static:
        out_type += [jax.ShapeDtypeStruct((NCORES, NPAD, D), jnp.float32)]

    def body(*refs):
        if include_static:
            (x_hbm, src_hbm, dst_hbm, est_hbm, z128_hbm,
             nbr_out, st_out,
             src_v, dst_v, buf0, buf1, acc, sem0, sem1) = refs
        else:
            (x_hbm, src_hbm, dst_hbm, z128_hbm, nbr_out,
             src_v, dst_v, buf0, buf1, acc, sem0, sem1) = refs

        c = lax.axis_index("c")
        s = lax.axis_index("s")
        w = c * NSUB + s
        base = w * CPW
        bufs = (buf0, buf1)
        sems = (sem0, sem1)

        def zero_acc():
            pltpu.sync_copy(z128_hbm.at[pl.ds(s * RPS, RPS)],
                            acc.at[pl.ds(s * RPS, RPS)])

        def writeout(out_ref):
            pltpu.sync_copy(acc.at[pl.ds(s * RPS, RPS)],
                            out_ref.at[c, pl.ds(s * RPS, RPS)])

        def edge_loop(load_src):
            @pl.loop(0, CPW // IBLK)
            def _(gb):
                pltpu.sync_copy(src_hbm.at[pl.ds(base + gb * IBLK, IBLK)],
                                src_v)
                pltpu.sync_copy(dst_hbm.at[pl.ds(base + gb * IBLK, IBLK)],
                                dst_v)
                load_src(gb, 0, buf0, sem0).start()
                for j in range(IBLK):
                    b = j % 2
                    load_src(gb, j, bufs[b], sems[b]).wait()
                    if j + 1 < IBLK:
                        load_src(gb, j + 1, bufs[1 - b], sems[1 - b]).start()
                    pltpu.sync_copy(bufs[b], acc.at[dst_v.at[j]], add=True)

        def gather_x(gb, j, buf, sem):
            return pltpu.make_async_copy(x_hbm.at[src_v.at[j]], buf, sem)

        zero_acc()
        plsc.subcore_barrier()
        edge_loop(gather_x)
        plsc.subcore_barrier()
        writeout(nbr_out)

        if include_static:
            def load_static(gb, j, buf, sem):
                row0 = (base + gb * IBLK + j) * CHUNK
                return pltpu.make_async_copy(
                    est_hbm.at[pl.ds(row0, CHUNK)], buf, sem)

            zero_acc()
            plsc.subcore_barrier()
            edge_loop(load_static)
            plsc.subcore_barrier()
            writeout(st_out)

    return pl.kernel(body, out_type=out_type, mesh=mesh, scratch_types=scratch)


def _sc_pass_full(*args):
    return _sc_edge_pass(True)(*args)


def _sc_pass_nbr(*args):
    return _sc_edge_pass(False)(*args)


def _est_body(srep, ea8, out):
    nr = EBLK // 8
    lane = lax.broadcasted_iota(jnp.int32, (nr, D), 1)
    sblk = srep[...]
    eblk = ea8[...]
    dbl_s = jnp.concatenate([sblk, sblk], axis=1)
    dbl_e = jnp.concatenate([eblk, eblk], axis=1)
    lanef = lane.astype(jnp.float32)
    for k in range(8):
        sv = dbl_s[:, DE * k:DE * k + D]
        ea = dbl_e[:, (DE * k - DE) % D:(DE * k - DE) % D + D]
        meta = jnp.where(lane == 0, 1.0,
                         jnp.where((lane >= 1) & (lane <= NSLOTS),
                                   (sv == lanef - 1.0).astype(jnp.float32),
                                   0.0))
        out[:, k, :] = jnp.where((lane >= DE) & (lane < 2 * DE), ea, meta)


def _build_est(slot_rep, ea8):
    row = lambda i: (i, 0)
    nr = EBLK // 8
    return pl.pallas_call(
        _est_body,
        grid=(EPAD // EBLK,),
        in_specs=[pl.BlockSpec((nr, D), row),
                  pl.BlockSpec((nr, D), row)],
        out_specs=pl.BlockSpec((nr, 8, D), lambda i: (i, 0, 0)),
        out_shape=jax.ShapeDtypeStruct((EPAD // 8, 8, D), jnp.float32),
    )(slot_rep, ea8).reshape(EPAD, D)


def _dup_body(x, out):
    out[...] = jnp.broadcast_to(x[...], (NCORES,) + x.shape)


def _dup_rows(x_pad):
    return pl.pallas_call(
        _dup_body,
        grid=(NPAD // TBLK,),
        in_specs=[pl.BlockSpec((TBLK, D), lambda i: (i, 0))],
        out_specs=pl.BlockSpec((NCORES, TBLK, D), lambda i: (0, i, 0)),
        out_shape=jax.ShapeDtypeStruct((NCORES, NPAD, D), jnp.float32),
    )(x_pad).reshape(NCORES * NPAD, D)


def _prep_body(u, Wm1, agg1, S1, bm1, wu1, Wm2, agg2, S2, bm2, wu2,
               B1o, T1o, B2o, T2o):
    hi = jax.lax.Precision.HIGHEST

    def one(Wm, agg, S, bm, wu, Bo, To):
        Bo[...] = jnp.dot(Wm[:D], agg[...], precision=hi)
        To[...] = jnp.concatenate([
            jnp.dot(bm[...], agg[...], precision=hi),
            jnp.dot(S[...], agg[...], precision=hi),
            jnp.zeros((ZROWS, D), jnp.float32),
            jnp.dot(Wm[D:], agg[...], precision=hi),
            jnp.dot(u[...], wu[...], precision=hi),
        ], axis=0)

    one(Wm1, agg1, S1, bm1, wu1, B1o, T1o)
    one(Wm2, agg2, S2, bm2, wu2, B2o, T2o)


def _node_body(has_residual, dup, x, nbrp, stp, batch, A, B, T, bias, res,
               out):
    hi = jax.lax.Precision.HIGHEST
    nbr = nbrp[0] + nbrp[1]
    st = stp[0] + stp[1]
    oh = (batch[...] == lax.broadcasted_iota(jnp.int32, (TBLK, NG), 1))
    z = jnp.concatenate([st[:, :2 * DE], oh.astype(jnp.float32)], axis=1)
    h = (jnp.dot(x[...], A[...], precision=hi)
         + jnp.dot(nbr, B[...], precision=hi)
         + jnp.dot(z, T[...], precision=hi)
         + bias[...])
    if has_residual:
        h = h + res[...]
    h = jnp.maximum(h, 0.0)
    if dup:
        out[...] = jnp.broadcast_to(h, (NCORES, TBLK, D))
    else:
        out[...] = h


def _node_layer(x_pad, nbr_p, st_p, batch_pad, A, B, T, bias, res_pad,
                has_residual, dup=False):
    nblk = NPAD // TBLK
    row = lambda i: (i, 0)
    full3 = pl.BlockSpec((NCORES, TBLK, D), lambda i: (0, i, 0))
    const = lambda shp: pl.BlockSpec(shp, lambda i: (0, 0))
    if dup:
        out_spec = pl.BlockSpec((NCORES, TBLK, D), lambda i: (0, i, 0))
        out_shape = jax.ShapeDtypeStruct((NCORES, NPAD, D), jnp.float32)
    else:
        out_spec = pl.BlockSpec((TBLK, D), row)
        out_shape = jax.ShapeDtypeStruct((NPAD, D), jnp.float32)
    return pl.pallas_call(
        functools.partial(_node_body, has_residual, dup),
        grid=(nblk,),
        in_specs=[
            pl.BlockSpec((TBLK, D), row),
            full3,
            full3,
            pl.BlockSpec((TBLK, 1), row),
            const((D, D)),
            const((D, D)),
            const((3 * 16, D)),
            const((1, D)),
            pl.BlockSpec((TBLK, D), row),
        ],
        out_specs=out_spec,
        out_shape=out_shape,
    )(x_pad, nbr_p, st_p, batch_pad, A, B, T, bias, res_pad)


def kernel(x, edge_index, edge_slot, edge_attr, u, batch,
           Wm1, bm1, S1, Wn1, bn1, Wm2, bm2, S2, Wn2, bn2):
    f32 = jnp.float32
    src = edge_index[0].astype(jnp.int32)
    dst = edge_index[1].astype(jnp.int32)
    slot = edge_slot.astype(jnp.int32)

    E = src.shape[0]
    pad_e = EPAD - E
    core_of_row = (jnp.arange(EPAD // CHUNK, dtype=jnp.int32)
                   // (CPW * NSUB))[:, None]
    src2d = (jnp.pad(src, (0, pad_e)).reshape(EPAD // CHUNK, CHUNK)
             + core_of_row * NPAD)
    dst2d = jnp.pad(dst, (0, pad_e), constant_values=DUMMY).reshape(
        EPAD // CHUNK, CHUNK)
    ea8 = jnp.pad(edge_attr.reshape(E // 8, D), ((0, pad_e // 8), (0, 0)))
    slot_rep = jnp.pad(
        jnp.broadcast_to(slot.astype(f32).reshape(E // 8, 8, 1),
                         (E // 8, 8, DE)).reshape(E // 8, D),
        ((0, pad_e // 8), (0, 0)))
    est128 = _build_est(slot_rep, ea8)

    x_pad = jnp.pad(x, ((0, NPAD - N), (0, 0)))
    batch_pad = jnp.pad(batch.astype(jnp.int32), (0, NPAD - N)).reshape(
        NPAD, 1)
    z128 = jnp.zeros((NPAD, D), f32)

    wshape = jax.ShapeDtypeStruct((D, D), f32)
    tshape = jax.ShapeDtypeStruct((3 * 16, D), f32)
    B1, T1, B2, T2 = pl.pallas_call(
        _prep_body,
        out_shape=[wshape, tshape, wshape, tshape],
    )(u, Wm1, Wn1[D:2 * D], S1, bm1.reshape(1, D), Wn1[2 * D:],
      Wm2, Wn2[D:2 * D], S2, bm2.reshape(1, D), Wn2[2 * D:])

    x_dup = _dup_rows(x_pad)
    nbr1_p, st_p = _sc_pass_full(x_dup, src2d, dst2d, est128, z128)

    h1_dup = _node_layer(x_pad, nbr1_p, st_p, batch_pad,
                         Wn1[:D], B1, T1, bn1.reshape(1, D), x_pad, False,
                         dup=True)

    (nbr2_p,) = _sc_pass_nbr(h1_dup.reshape(NCORES * NPAD, D), src2d, dst2d,
                             z128)

    out = _node_layer(h1_dup[0], nbr2_p, st_p, batch_pad,
                      Wn2[:D], B2, T2, bn2.reshape(1, D), x_pad, True)

    return out[:N]
